# Initial kernel scaffold; baseline (speedup 1.0000x reference)
import jax
import jax.numpy as jnp
from jax.experimental import pallas as pl


def kernel(router_outputs, attention_mask):
    raise NotImplementedError("write your pallas kernel here")


# speedup vs baseline: 6.5362x; 6.5362x over previous
import functools
import math

import numpy as np
import jax
import jax.numpy as jnp
from jax import lax
from jax.experimental import pallas as pl
from jax.experimental.pallas import tpu as pltpu
from jax.experimental.pallas import tpu_sc as plsc

_MIU = 32.0
_ROUTE_COEF = 0.001
_COUNT_COEF = 0.0001

_L = 16
_T = 8192
_E = 64
_TP3 = _T + 3
_NROWS = _L * _T
_NW = 32
_TOK_PER_W = _T // 2
_CHUNK = 512
_NCHUNK = _TOK_PER_W // _CHUNK
_NGROUP = _CHUNK // 16

_LN2 = np.float32(math.log(2.0))
_LGAMMA_KP1 = np.asarray([math.lgamma(k + 1.0) for k in range(1, _E + 1)],
                         np.float32)


def _oddeven_merge(lo, hi, r):
    step = r * 2
    if step < hi - lo:
        yield from _oddeven_merge(lo, hi, step)
        yield from _oddeven_merge(lo + r, hi, step)
        for i in range(lo + r, hi - r, step):
            yield (i, i + r)
    else:
        yield (lo, lo + r)


def _oddeven_merge_sort_range(lo, hi):
    if (hi - lo) >= 1:
        mid = lo + ((hi - lo) // 2)
        yield from _oddeven_merge_sort_range(lo, mid)
        yield from _oddeven_merge_sort_range(mid + 1, hi)
        yield from _oddeven_merge(lo, hi, 1)


_SORT16_CES = tuple(_oddeven_merge_sort_range(0, 15))


def _pois_body(lamb_ref, lgam_ref, a_ref, b_ref):
    lamb = jnp.clip(lamb_ref[...], 1.0, 2.0)
    kv = lax.broadcasted_iota(jnp.int32, (_L, _E), 1).astype(jnp.float32) + 1.0
    lgam = lgam_ref[...]
    logp = kv * jnp.log(lamb) - lamb - lgam
    ppu = jnp.exp(logp)
    col = lax.broadcasted_iota(jnp.int32, (_L, _E), 1)
    p31 = lax.broadcast_in_dim(ppu[:, 31:32], (_L, _E), (0, 1))
    ppu = jnp.where(col >= 32, p31, ppu)
    pp = ppu / jnp.sum(ppu, axis=1, keepdims=True)
    a = pp[:, :16]
    b = a * jnp.log(a)
    a_ref[...] = lax.broadcast_in_dim(a, (_L, 16, 16), (0, 1))
    b_ref[...] = lax.broadcast_in_dim(b, (_L, 16, 16), (0, 1))


_pois_call = pl.pallas_call(
    _pois_body,
    out_shape=[jax.ShapeDtypeStruct((_L, 16, 16), jnp.float32),
               jax.ShapeDtypeStruct((_L, 16, 16), jnp.float32)],
)


def _sc_log(x):
    bits = lax.bitcast_convert_type(x, jnp.int32)
    e = jnp.bitwise_and(lax.shift_right_logical(bits, 23), 0xFF)
    mb = jnp.bitwise_or(jnp.bitwise_and(bits, 0x007FFFFF), 0x3F800000)
    m = lax.bitcast_convert_type(mb, jnp.float32)
    big = m > 1.5
    m = jnp.where(big, m * 0.5, m)
    ef = (e - 127).astype(jnp.float32) + jnp.where(big, 1.0, 0.0)
    s = (m - 1.0) / (m + 1.0)
    t = s * s
    lm = s * (2.0 + t * (np.float32(2.0 / 3.0)
              + t * (np.float32(2.0 / 5.0)
              + t * (np.float32(2.0 / 7.0)
              + t * np.float32(2.0 / 9.0)))))
    return ef * _LN2 + lm


def _sort16_desc(vals):
    vals = list(vals)
    for i, j in _SORT16_CES:
        hi = jnp.maximum(vals[i], vals[j])
        lo = jnp.minimum(vals[i], vals[j])
        vals[i], vals[j] = hi, lo
    return vals


def _merge_top16(a, b):
    c = [jnp.maximum(a[i], b[15 - i]) for i in range(16)]
    for d in (8, 4, 2, 1):
        for i in range(16):
            if (i % (2 * d)) < d:
                hi = jnp.maximum(c[i], c[i + d])
                lo = jnp.minimum(c[i], c[i + d])
                c[i], c[i + d] = hi, lo
    return c


def _route_body(x_hbm, a_hbm, b_hbm, out_hbm, buf, abuf, bbuf, accv):
    cid = lax.axis_index("c")
    sid = lax.axis_index("s")
    wid = sid * 2 + cid
    layer = wid // 2
    t_base = (wid % 2) * _TOK_PER_W

    pltpu.sync_copy(a_hbm.at[layer], abuf)
    pltpu.sync_copy(b_hbm.at[layer], bbuf)

    def process(buf, acc):
        def group_step(g, acc):
            t0 = pl.multiple_of(g * 16, 16)
            regs = [buf[e, pl.ds(t0, 16)] for e in range(_E)]
            s0 = _sort16_desc(regs[0:16])
            s1 = _sort16_desc(regs[16:32])
            m1 = _merge_top16(s0, s1)
            s2 = _sort16_desc(regs[32:48])
            s3 = _sort16_desc(regs[48:64])
            m2 = _merge_top16(s2, s3)
            top = _merge_top16(m1, m2)
            for k in range(16):
                lx = _sc_log(top[k] + 1e-16)
                acc = acc + jnp.abs(bbuf[k, :] - abuf[k, :] * lx)
            return acc
        return lax.fori_loop(0, _NGROUP, group_step, acc)

    def chunk_body(ci, acc):
        pltpu.sync_copy(
            x_hbm.at[layer, :, pl.ds(t_base + ci * _CHUNK, _CHUNK)], buf)
        return process(buf, acc)

    acc = lax.fori_loop(0, _NCHUNK, chunk_body, jnp.zeros((16,), jnp.float32))
    accv[...] = acc
    pltpu.sync_copy(accv, out_hbm.at[wid])


def _make_route_call():
    mesh = plsc.VectorSubcoreMesh(core_axis_name="c", subcore_axis_name="s")
    return functools.partial(
        pl.kernel,
        out_type=jax.ShapeDtypeStruct((_NW, 16), jnp.float32),
        mesh=mesh,
        scratch_types=[
            pltpu.VMEM((_E, _CHUNK), jnp.float32),
            pltpu.VMEM((16, 16), jnp.float32),
            pltpu.VMEM((16, 16), jnp.float32),
            pltpu.VMEM((16,), jnp.float32),
        ],
        compiler_params=pltpu.CompilerParams(use_tc_tiling_on_sc=True,
                                             needs_layout_passes=False),
    )(_route_body)


def _sort_desc_rows(v):
    vi = v[:, :, None]
    vj = v[:, None, :]
    ii = lax.broadcasted_iota(jnp.int32, (_L, _E, _E), 1)
    jj = lax.broadcasted_iota(jnp.int32, (_L, _E, _E), 2)
    gt = (vj > vi) | ((vj == vi) & (jj < ii))
    rank = jnp.sum(gt.astype(jnp.int32), axis=2)
    rr = lax.broadcasted_iota(jnp.int32, (_L, _E, _E), 2)
    onehot = rank[:, :, None] == rr
    return jnp.sum(jnp.where(onehot, vi, 0.0), axis=1)


def _final_body(count_ref, theta_ref, part_ref, out_ref):
    count = count_ref[...]
    theta = theta_ref[...]
    x = lax.broadcasted_iota(jnp.int32, (_L, _E), 1).astype(jnp.float32)
    lp = (-0.5 * ((x - _MIU) / theta) ** 2 - jnp.log(theta)
          - np.float32(0.5 * math.log(2.0 * math.pi)))
    norm_prob = jnp.exp(lp)
    norm_prob = norm_prob / jnp.sum(norm_prob)
    s_count = _sort_desc_rows(count)
    s_np = _sort_desc_rows(norm_prob)
    temp = s_count / jnp.sum(s_count, axis=1, keepdims=True)
    lterm = s_np * jnp.abs(jnp.log(s_np / (temp + 1e-8)))
    closs = jnp.sum(lterm) / np.float32(_L)
    route = jnp.sum(part_ref[...]) / np.float32(_NROWS + 1e-16)
    res = _ROUTE_COEF * route + _COUNT_COEF * closs
    out_ref[...] = lax.broadcast_in_dim(res, (1, 1), ())


_final_call = pl.pallas_call(
    _final_body,
    out_shape=jax.ShapeDtypeStruct((1, 1), jnp.float32),
)


def kernel(router_outputs, attention_mask):
    del attention_mask
    count = router_outputs[:, -3, :]
    lamb = router_outputs[:, -2, :]
    theta = router_outputs[:, -1, :]
    a, b = _pois_call(lamb, jnp.asarray(_LGAMMA_KP1)[None, :])
    xt = jnp.swapaxes(router_outputs, 1, 2)
    partials = _make_route_call()(xt, a, b)
    out = _final_call(count, theta, partials)
    return out[0, 0]


# speedup vs baseline: 8.0301x; 1.2286x over previous
import functools
import math

import numpy as np
import jax
import jax.numpy as jnp
from jax import lax
from jax.experimental import pallas as pl
from jax.experimental.pallas import tpu as pltpu
from jax.experimental.pallas import tpu_sc as plsc

_MIU = 32.0
_ROUTE_COEF = 0.001
_COUNT_COEF = 0.0001

_L = 16
_T = 8192
_E = 64
_TP3 = _T + 3
_NROWS = _L * _T
_NW = 32

_SC_TOK = 2048
_TC_TOK = _T - _SC_TOK
_TC_BLK = 1024
_TC_NBLK = _TC_TOK // _TC_BLK

_TOK_PER_W = _SC_TOK * _L // _NW
_CHUNK = 512
_NCHUNK = _TOK_PER_W // _CHUNK
_NGROUP = _CHUNK // 16

_LN2 = np.float32(math.log(2.0))
_LGAMMA_KP1 = np.asarray([math.lgamma(k + 1.0) for k in range(1, _E + 1)],
                         np.float32)


def _oddeven_merge(lo, hi, r):
    step = r * 2
    if step < hi - lo:
        yield from _oddeven_merge(lo, hi, step)
        yield from _oddeven_merge(lo + r, hi, step)
        for i in range(lo + r, hi - r, step):
            yield (i, i + r)
    else:
        yield (lo, lo + r)


def _oddeven_merge_sort_range(lo, hi):
    if (hi - lo) >= 1:
        mid = lo + ((hi - lo) // 2)
        yield from _oddeven_merge_sort_range(lo, mid)
        yield from _oddeven_merge_sort_range(mid + 1, hi)
        yield from _oddeven_merge(lo, hi, 1)


_SORT16_CES = tuple(_oddeven_merge_sort_range(0, 15))


def _pois_body(lamb_ref, lgam_ref, a_ref, b_ref):
    lamb = jnp.clip(lamb_ref[...], 1.0, 2.0)
    kv = lax.broadcasted_iota(jnp.int32, (_L, _E), 1).astype(jnp.float32) + 1.0
    lgam = lgam_ref[...]
    logp = kv * jnp.log(lamb) - lamb - lgam
    ppu = jnp.exp(logp)
    col = lax.broadcasted_iota(jnp.int32, (_L, _E), 1)
    p31 = lax.broadcast_in_dim(ppu[:, 31:32], (_L, _E), (0, 1))
    ppu = jnp.where(col >= 32, p31, ppu)
    pp = ppu / jnp.sum(ppu, axis=1, keepdims=True)
    a = pp[:, :16]
    b = a * jnp.log(a)
    a_ref[...] = lax.broadcast_in_dim(a, (_L, 16, 16), (0, 1))
    b_ref[...] = lax.broadcast_in_dim(b, (_L, 16, 16), (0, 1))


_pois_call = pl.pallas_call(
    _pois_body,
    out_shape=[jax.ShapeDtypeStruct((_L, 16, 16), jnp.float32),
               jax.ShapeDtypeStruct((_L, 16, 16), jnp.float32)],
)


def _sc_log(x):
    bits = lax.bitcast_convert_type(x, jnp.int32)
    e = jnp.bitwise_and(lax.shift_right_logical(bits, 23), 0xFF)
    mb = jnp.bitwise_or(jnp.bitwise_and(bits, 0x007FFFFF), 0x3F800000)
    m = lax.bitcast_convert_type(mb, jnp.float32)
    big = m > 1.5
    m = jnp.where(big, m * 0.5, m)
    ef = (e - 127).astype(jnp.float32) + jnp.where(big, 1.0, 0.0)
    s = (m - 1.0) / (m + 1.0)
    t = s * s
    lm = s * (2.0 + t * (np.float32(2.0 / 3.0)
              + t * (np.float32(2.0 / 5.0)
              + t * (np.float32(2.0 / 7.0)
              + t * np.float32(2.0 / 9.0)))))
    return ef * _LN2 + lm


def _sort16_desc(vals):
    vals = list(vals)
    for i, j in _SORT16_CES:
        hi = jnp.maximum(vals[i], vals[j])
        lo = jnp.minimum(vals[i], vals[j])
        vals[i], vals[j] = hi, lo
    return vals


def _merge_top16(a, b):
    c = [jnp.maximum(a[i], b[15 - i]) for i in range(16)]
    for d in (8, 4, 2, 1):
        for i in range(16):
            if (i % (2 * d)) < d:
                hi = jnp.maximum(c[i], c[i + d])
                lo = jnp.minimum(c[i], c[i + d])
                c[i], c[i + d] = hi, lo
    return c


def _route_body(x_hbm, a_hbm, b_hbm, out_hbm, buf, abuf, bbuf, accv):
    cid = lax.axis_index("c")
    sid = lax.axis_index("s")
    wid = sid * 2 + cid
    layer = wid // 2
    t_base = _TC_TOK + (wid % 2) * _TOK_PER_W

    pltpu.sync_copy(a_hbm.at[layer], abuf)
    pltpu.sync_copy(b_hbm.at[layer], bbuf)

    def process(buf, acc):
        def group_step(g, acc):
            t0 = pl.multiple_of(g * 16, 16)
            regs = [buf[e, pl.ds(t0, 16)] for e in range(_E)]
            s0 = _sort16_desc(regs[0:16])
            s1 = _sort16_desc(regs[16:32])
            m1 = _merge_top16(s0, s1)
            s2 = _sort16_desc(regs[32:48])
            s3 = _sort16_desc(regs[48:64])
            m2 = _merge_top16(s2, s3)
            top = _merge_top16(m1, m2)
            terms = [jnp.abs(bbuf[k, :] - abuf[k, :] * _sc_log(top[k] + 1e-16))
                     for k in range(16)]
            while len(terms) > 1:
                terms = [terms[i] + terms[i + 1]
                         for i in range(0, len(terms), 2)]
            return acc + terms[0]
        return lax.fori_loop(0, _NGROUP, group_step, acc)

    def chunk_body(ci, acc):
        pltpu.sync_copy(
            x_hbm.at[layer, :, pl.ds(t_base + ci * _CHUNK, _CHUNK)], buf)
        return process(buf, acc)

    acc = lax.fori_loop(0, _NCHUNK, chunk_body, jnp.zeros((16,), jnp.float32))
    accv[...] = acc
    pltpu.sync_copy(accv, out_hbm.at[wid])


def _make_route_call():
    mesh = plsc.VectorSubcoreMesh(core_axis_name="c", subcore_axis_name="s")
    return functools.partial(
        pl.kernel,
        out_type=jax.ShapeDtypeStruct((_NW, 16), jnp.float32),
        mesh=mesh,
        scratch_types=[
            pltpu.VMEM((_E, _CHUNK), jnp.float32),
            pltpu.VMEM((16, 16), jnp.float32),
            pltpu.VMEM((16, 16), jnp.float32),
            pltpu.VMEM((16,), jnp.float32),
        ],
        compiler_params=pltpu.CompilerParams(use_tc_tiling_on_sc=True,
                                             needs_layout_passes=False),
    )(_route_body)


def _tc_route_body(a_ref, b_ref, x_ref, out_ref):
    l = pl.program_id(0)
    x = x_ref[0]
    x3 = x.reshape(_E, _TC_BLK // 128, 128)
    regs = [x3[e] for e in range(_E)]
    s0 = _sort16_desc(regs[0:16])
    s1 = _sort16_desc(regs[16:32])
    m1 = _merge_top16(s0, s1)
    s2 = _sort16_desc(regs[32:48])
    s3 = _sort16_desc(regs[48:64])
    m2 = _merge_top16(s2, s3)
    top = _merge_top16(m1, m2)
    terms = [jnp.abs(b_ref[l, k, 0] - a_ref[l, k, 0] * jnp.log(top[k] + 1e-16))
             for k in range(16)]
    while len(terms) > 1:
        terms = [terms[i] + terms[i + 1] for i in range(0, len(terms), 2)]
    out_ref[0, 0] = terms[0]


def _tc_route_call(a, b, xt):
    return pl.pallas_call(
        _tc_route_body,
        grid=(_L, _TC_NBLK),
        in_specs=[
            pl.BlockSpec(memory_space=pltpu.SMEM),
            pl.BlockSpec(memory_space=pltpu.SMEM),
            pl.BlockSpec((1, _E, _TC_BLK), lambda l, t: (l, 0, t)),
        ],
        out_specs=pl.BlockSpec((1, 1, _TC_BLK // 128, 128),
                               lambda l, t: (l, t, 0, 0)),
        out_shape=jax.ShapeDtypeStruct((_L, _TC_NBLK, _TC_BLK // 128, 128),
                                       jnp.float32),
    )(a, b, xt)


def _sort_desc_rows(v):
    vi = v[:, :, None]
    vj = v[:, None, :]
    ii = lax.broadcasted_iota(jnp.int32, (_L, _E, _E), 1)
    jj = lax.broadcasted_iota(jnp.int32, (_L, _E, _E), 2)
    gt = (vj > vi) | ((vj == vi) & (jj < ii))
    rank = jnp.sum(gt.astype(jnp.int32), axis=2)
    rr = lax.broadcasted_iota(jnp.int32, (_L, _E, _E), 2)
    onehot = rank[:, :, None] == rr
    return jnp.sum(jnp.where(onehot, vi, 0.0), axis=1)


def _final_body(count_ref, theta_ref, part_ref, tcpart_ref, out_ref):
    count = count_ref[...]
    theta = theta_ref[...]
    x = lax.broadcasted_iota(jnp.int32, (_L, _E), 1).astype(jnp.float32)
    lp = (-0.5 * ((x - _MIU) / theta) ** 2 - jnp.log(theta)
          - np.float32(0.5 * math.log(2.0 * math.pi)))
    norm_prob = jnp.exp(lp)
    norm_prob = norm_prob / jnp.sum(norm_prob)
    s_count = _sort_desc_rows(count)
    s_np = _sort_desc_rows(norm_prob)
    temp = s_count / jnp.sum(s_count, axis=1, keepdims=True)
    lterm = s_np * jnp.abs(jnp.log(s_np / (temp + 1e-8)))
    closs = jnp.sum(lterm) / np.float32(_L)
    route = ((jnp.sum(part_ref[...]) + jnp.sum(tcpart_ref[...]))
             / np.float32(_NROWS + 1e-16))
    res = _ROUTE_COEF * route + _COUNT_COEF * closs
    out_ref[...] = lax.broadcast_in_dim(res, (1, 1), ())


_final_call = pl.pallas_call(
    _final_body,
    out_shape=jax.ShapeDtypeStruct((1, 1), jnp.float32),
)


def kernel(router_outputs, attention_mask):
    del attention_mask
    count = router_outputs[:, -3, :]
    lamb = router_outputs[:, -2, :]
    theta = router_outputs[:, -1, :]
    a, b = _pois_call(lamb, jnp.asarray(_LGAMMA_KP1)[None, :])
    xt = jnp.swapaxes(router_outputs, 1, 2)
    partials = _make_route_call()(xt, a, b)
    tc_partials = _tc_route_call(a, b, xt)
    out = _final_call(count, theta, partials, tc_partials)
    return out[0, 0]


# speedup vs baseline: 10.9116x; 1.3588x over previous
import functools
import math

import numpy as np
import jax
import jax.numpy as jnp
from jax import lax
from jax.experimental import pallas as pl
from jax.experimental.pallas import tpu as pltpu
from jax.experimental.pallas import tpu_sc as plsc

_MIU = 32.0
_ROUTE_COEF = 0.001
_COUNT_COEF = 0.0001

_L = 16
_T = 8192
_E = 64
_TP3 = _T + 3
_NROWS = _L * _T
_NW = 32

_SC_TOK = 2048
_TC_TOK = _T - _SC_TOK
_TC_BLK = 2048
_TC_NBLK = _TC_TOK // _TC_BLK

_TOK_PER_W = _SC_TOK * _L // _NW
_CHUNK = 512
_NCHUNK = _TOK_PER_W // _CHUNK
_NGROUP = _CHUNK // 16

_LN2 = np.float32(math.log(2.0))
_LGAMMA_KP1 = np.asarray([math.lgamma(k + 1.0) for k in range(1, _E + 1)],
                         np.float32)


def _oddeven_merge(lo, hi, r):
    step = r * 2
    if step < hi - lo:
        yield from _oddeven_merge(lo, hi, step)
        yield from _oddeven_merge(lo + r, hi, step)
        for i in range(lo + r, hi - r, step):
            yield (i, i + r)
    else:
        yield (lo, lo + r)


def _oddeven_merge_sort_range(lo, hi):
    if (hi - lo) >= 1:
        mid = lo + ((hi - lo) // 2)
        yield from _oddeven_merge_sort_range(lo, mid)
        yield from _oddeven_merge_sort_range(mid + 1, hi)
        yield from _oddeven_merge(lo, hi, 1)


_SORT16_CES = tuple(_oddeven_merge_sort_range(0, 15))


def _pois_body(lamb_ref, lgam_ref, a_ref, b_ref):
    lamb = jnp.clip(lamb_ref[...], 1.0, 2.0)
    kv = lax.broadcasted_iota(jnp.int32, (_L, _E), 1).astype(jnp.float32) + 1.0
    lgam = lgam_ref[...]
    logp = kv * jnp.log(lamb) - lamb - lgam
    ppu = jnp.exp(logp)
    col = lax.broadcasted_iota(jnp.int32, (_L, _E), 1)
    p31 = lax.broadcast_in_dim(ppu[:, 31:32], (_L, _E), (0, 1))
    ppu = jnp.where(col >= 32, p31, ppu)
    pp = ppu / jnp.sum(ppu, axis=1, keepdims=True)
    a = pp[:, :16]
    b = a * jnp.log(a)
    a_ref[...] = lax.broadcast_in_dim(a, (_L, 16, 16), (0, 1))
    b_ref[...] = lax.broadcast_in_dim(b, (_L, 16, 16), (0, 1))


_pois_call = pl.pallas_call(
    _pois_body,
    out_shape=[jax.ShapeDtypeStruct((_L, 16, 16), jnp.float32),
               jax.ShapeDtypeStruct((_L, 16, 16), jnp.float32)],
)


def _sc_log(x):
    bits = lax.bitcast_convert_type(x, jnp.int32)
    e = jnp.bitwise_and(lax.shift_right_logical(bits, 23), 0xFF)
    mb = jnp.bitwise_or(jnp.bitwise_and(bits, 0x007FFFFF), 0x3F800000)
    m = lax.bitcast_convert_type(mb, jnp.float32)
    big = m > 1.5
    m = jnp.where(big, m * 0.5, m)
    ef = (e - 127).astype(jnp.float32) + jnp.where(big, 1.0, 0.0)
    s = (m - 1.0) / (m + 1.0)
    t = s * s
    lm = s * (2.0 + t * (np.float32(2.0 / 3.0)
              + t * (np.float32(2.0 / 5.0)
              + t * (np.float32(2.0 / 7.0)
              + t * np.float32(2.0 / 9.0)))))
    return ef * _LN2 + lm


def _sort16_desc(vals):
    vals = list(vals)
    for i, j in _SORT16_CES:
        hi = jnp.maximum(vals[i], vals[j])
        lo = jnp.minimum(vals[i], vals[j])
        vals[i], vals[j] = hi, lo
    return vals


def _merge_top16(a, b):
    c = [jnp.maximum(a[i], b[15 - i]) for i in range(16)]
    for d in (8, 4, 2, 1):
        for i in range(16):
            if (i % (2 * d)) < d:
                hi = jnp.maximum(c[i], c[i + d])
                lo = jnp.minimum(c[i], c[i + d])
                c[i], c[i + d] = hi, lo
    return c


def _route_body(x_hbm, a_hbm, b_hbm, out_hbm, buf, abuf, bbuf, accv):
    cid = lax.axis_index("c")
    sid = lax.axis_index("s")
    wid = sid * 2 + cid
    layer = wid // 2
    t_base = _TC_TOK + (wid % 2) * _TOK_PER_W

    pltpu.sync_copy(a_hbm.at[layer], abuf)
    pltpu.sync_copy(b_hbm.at[layer], bbuf)

    def process(buf, acc):
        def group_step(g, acc):
            t0 = pl.multiple_of(g * 16, 16)
            regs = [buf[e, pl.ds(t0, 16)] for e in range(_E)]
            s0 = _sort16_desc(regs[0:16])
            s1 = _sort16_desc(regs[16:32])
            m1 = _merge_top16(s0, s1)
            s2 = _sort16_desc(regs[32:48])
            s3 = _sort16_desc(regs[48:64])
            m2 = _merge_top16(s2, s3)
            top = _merge_top16(m1, m2)
            terms = [jnp.abs(bbuf[k, :] - abuf[k, :] * _sc_log(top[k] + 1e-16))
                     for k in range(16)]
            while len(terms) > 1:
                terms = [terms[i] + terms[i + 1]
                         for i in range(0, len(terms), 2)]
            return acc + terms[0]
        return lax.fori_loop(0, _NGROUP, group_step, acc)

    def chunk_body(ci, acc):
        pltpu.sync_copy(
            x_hbm.at[layer, :, pl.ds(t_base + ci * _CHUNK, _CHUNK)], buf)
        return process(buf, acc)

    acc = lax.fori_loop(0, _NCHUNK, chunk_body, jnp.zeros((16,), jnp.float32))
    accv[...] = acc
    pltpu.sync_copy(accv, out_hbm.at[wid])


def _make_route_call():
    mesh = plsc.VectorSubcoreMesh(core_axis_name="c", subcore_axis_name="s")
    return functools.partial(
        pl.kernel,
        out_type=jax.ShapeDtypeStruct((_NW, 16), jnp.float32),
        mesh=mesh,
        scratch_types=[
            pltpu.VMEM((_E, _CHUNK), jnp.float32),
            pltpu.VMEM((16, 16), jnp.float32),
            pltpu.VMEM((16, 16), jnp.float32),
            pltpu.VMEM((16,), jnp.float32),
        ],
        compiler_params=pltpu.CompilerParams(use_tc_tiling_on_sc=True,
                                             needs_layout_passes=False),
    )(_route_body)


def _tc_route_body(a_ref, b_ref, x_ref, out_ref):
    l = pl.program_id(0)
    x = x_ref[0]
    x3 = x.reshape(_E, _TC_BLK // 128, 128)
    regs = [x3[e] for e in range(_E)]
    s0 = _sort16_desc(regs[0:16])
    s1 = _sort16_desc(regs[16:32])
    m1 = _merge_top16(s0, s1)
    s2 = _sort16_desc(regs[32:48])
    s3 = _sort16_desc(regs[48:64])
    m2 = _merge_top16(s2, s3)
    top = _merge_top16(m1, m2)
    terms = [jnp.abs(b_ref[l, k, 0] - a_ref[l, k, 0] * jnp.log(top[k] + 1e-16))
             for k in range(16)]
    while len(terms) > 1:
        terms = [terms[i] + terms[i + 1] for i in range(0, len(terms), 2)]
    out_ref[0, 0] = terms[0]


def _tc_route_call(a, b, xt):
    return pl.pallas_call(
        _tc_route_body,
        grid=(_L, _TC_NBLK),
        in_specs=[
            pl.BlockSpec(memory_space=pltpu.SMEM),
            pl.BlockSpec(memory_space=pltpu.SMEM),
            pl.BlockSpec((1, _E, _TC_BLK), lambda l, t: (l, 0, t)),
        ],
        out_specs=pl.BlockSpec((1, 1, _TC_BLK // 128, 128),
                               lambda l, t: (l, t, 0, 0)),
        out_shape=jax.ShapeDtypeStruct((_L, _TC_NBLK, _TC_BLK // 128, 128),
                                       jnp.float32),
    )(a, b, xt)


def _sort_desc_rows(v):
    vi = v[:, :, None]
    vj = v[:, None, :]
    ii = lax.broadcasted_iota(jnp.int32, (_L, _E, _E), 1)
    jj = lax.broadcasted_iota(jnp.int32, (_L, _E, _E), 2)
    gt = (vj > vi) | ((vj == vi) & (jj < ii))
    rank = jnp.sum(gt.astype(jnp.int32), axis=2)
    rr = lax.broadcasted_iota(jnp.int32, (_L, _E, _E), 2)
    onehot = rank[:, :, None] == rr
    return jnp.sum(jnp.where(onehot, vi, 0.0), axis=1)


def _final_body(count_ref, theta_ref, part_ref, tcpart_ref, out_ref):
    count = count_ref[...]
    theta = theta_ref[...]
    x = lax.broadcasted_iota(jnp.int32, (_L, _E), 1).astype(jnp.float32)
    lp = (-0.5 * ((x - _MIU) / theta) ** 2 - jnp.log(theta)
          - np.float32(0.5 * math.log(2.0 * math.pi)))
    norm_prob = jnp.exp(lp)
    norm_prob = norm_prob / jnp.sum(norm_prob)
    s_count = _sort_desc_rows(count)
    s_np = _sort_desc_rows(norm_prob)
    temp = s_count / jnp.sum(s_count, axis=1, keepdims=True)
    lterm = s_np * jnp.abs(jnp.log(s_np / (temp + 1e-8)))
    closs = jnp.sum(lterm) / np.float32(_L)
    route = ((jnp.sum(part_ref[...]) + jnp.sum(tcpart_ref[...]))
             / np.float32(_NROWS + 1e-16))
    res = _ROUTE_COEF * route + _COUNT_COEF * closs
    out_ref[...] = lax.broadcast_in_dim(res, (1, 1), ())


_final_call = pl.pallas_call(
    _final_body,
    out_shape=jax.ShapeDtypeStruct((1, 1), jnp.float32),
)


def kernel(router_outputs, attention_mask):
    del attention_mask
    count = router_outputs[:, -3, :]
    lamb = router_outputs[:, -2, :]
    theta = router_outputs[:, -1, :]
    a, b = _pois_call(lamb, jnp.asarray(_LGAMMA_KP1)[None, :])
    xt = jnp.swapaxes(router_outputs, 1, 2)
    partials = _make_route_call()(xt, a, b)
    tc_partials = _tc_route_call(a, b, xt)
    out = _final_call(count, theta, partials, tc_partials)
    return out[0, 0]


# speedup vs baseline: 12.3943x; 1.1359x over previous
import functools
import math

import numpy as np
import jax
import jax.numpy as jnp
from jax import lax
from jax.experimental import pallas as pl
from jax.experimental.pallas import tpu as pltpu
from jax.experimental.pallas import tpu_sc as plsc

_MIU = 32.0
_ROUTE_COEF = 0.001
_COUNT_COEF = 0.0001

_L = 16
_T = 8192
_E = 64
_TP3 = _T + 3
_NROWS = _L * _T
_NW = 32

_SC_TOK = 2048
_TC_TOK = _T - _SC_TOK
_TC_BLK = 3072
_TC_NBLK = _TC_TOK // _TC_BLK

_TOK_PER_W = _SC_TOK * _L // _NW
_CHUNK = 512
_NCHUNK = _TOK_PER_W // _CHUNK
_NGROUP = _CHUNK // 16

_LN2 = np.float32(math.log(2.0))
_LGAMMA_KP1 = np.asarray([math.lgamma(k + 1.0) for k in range(1, _E + 1)],
                         np.float32)


def _oddeven_merge(lo, hi, r):
    step = r * 2
    if step < hi - lo:
        yield from _oddeven_merge(lo, hi, step)
        yield from _oddeven_merge(lo + r, hi, step)
        for i in range(lo + r, hi - r, step):
            yield (i, i + r)
    else:
        yield (lo, lo + r)


def _oddeven_merge_sort_range(lo, hi):
    if (hi - lo) >= 1:
        mid = lo + ((hi - lo) // 2)
        yield from _oddeven_merge_sort_range(lo, mid)
        yield from _oddeven_merge_sort_range(mid + 1, hi)
        yield from _oddeven_merge(lo, hi, 1)


_SORT16_CES = tuple(_oddeven_merge_sort_range(0, 15))


def _pois_body(lamb_ref, lgam_ref, a_ref, b_ref):
    lamb = jnp.clip(lamb_ref[...], 1.0, 2.0)
    kv = lax.broadcasted_iota(jnp.int32, (_L, _E), 1).astype(jnp.float32) + 1.0
    lgam = lgam_ref[...]
    logp = kv * jnp.log(lamb) - lamb - lgam
    ppu = jnp.exp(logp)
    col = lax.broadcasted_iota(jnp.int32, (_L, _E), 1)
    p31 = lax.broadcast_in_dim(ppu[:, 31:32], (_L, _E), (0, 1))
    ppu = jnp.where(col >= 32, p31, ppu)
    pp = ppu / jnp.sum(ppu, axis=1, keepdims=True)
    a = pp[:, :16]
    b = a * jnp.log(a)
    a_ref[...] = lax.broadcast_in_dim(a, (_L, 16, 16), (0, 1))
    b_ref[...] = lax.broadcast_in_dim(b, (_L, 16, 16), (0, 1))


_pois_call = pl.pallas_call(
    _pois_body,
    out_shape=[jax.ShapeDtypeStruct((_L, 16, 16), jnp.float32),
               jax.ShapeDtypeStruct((_L, 16, 16), jnp.float32)],
)


def _sc_log(x):
    bits = lax.bitcast_convert_type(x, jnp.int32)
    e = jnp.bitwise_and(lax.shift_right_logical(bits, 23), 0xFF)
    mb = jnp.bitwise_or(jnp.bitwise_and(bits, 0x007FFFFF), 0x3F800000)
    m = lax.bitcast_convert_type(mb, jnp.float32)
    big = m > 1.5
    m = jnp.where(big, m * 0.5, m)
    ef = (e - 127).astype(jnp.float32) + jnp.where(big, 1.0, 0.0)
    s = (m - 1.0) / (m + 1.0)
    t = s * s
    lm = s * (2.0 + t * (np.float32(2.0 / 3.0)
              + t * (np.float32(2.0 / 5.0)
              + t * (np.float32(2.0 / 7.0)
              + t * np.float32(2.0 / 9.0)))))
    return ef * _LN2 + lm


def _sort16_desc(vals):
    vals = list(vals)
    for i, j in _SORT16_CES:
        hi = jnp.maximum(vals[i], vals[j])
        lo = jnp.minimum(vals[i], vals[j])
        vals[i], vals[j] = hi, lo
    return vals


def _merge_top16(a, b):
    c = [jnp.maximum(a[i], b[15 - i]) for i in range(16)]
    for d in (8, 4, 2, 1):
        for i in range(16):
            if (i % (2 * d)) < d:
                hi = jnp.maximum(c[i], c[i + d])
                lo = jnp.minimum(c[i], c[i + d])
                c[i], c[i + d] = hi, lo
    return c


def _route_body(x_hbm, a_hbm, b_hbm, out_hbm, buf, abuf, bbuf, accv):
    cid = lax.axis_index("c")
    sid = lax.axis_index("s")
    wid = sid * 2 + cid
    layer = wid // 2
    t_base = _TC_TOK + (wid % 2) * _TOK_PER_W

    pltpu.sync_copy(a_hbm.at[layer], abuf)
    pltpu.sync_copy(b_hbm.at[layer], bbuf)

    def process(buf, acc):
        def group_step(g, acc):
            t0 = pl.multiple_of(g * 16, 16)
            regs = [buf[e, pl.ds(t0, 16)] for e in range(_E)]
            s0 = _sort16_desc(regs[0:16])
            s1 = _sort16_desc(regs[16:32])
            m1 = _merge_top16(s0, s1)
            s2 = _sort16_desc(regs[32:48])
            s3 = _sort16_desc(regs[48:64])
            m2 = _merge_top16(s2, s3)
            top = _merge_top16(m1, m2)
            terms = [jnp.abs(bbuf[k, :] - abuf[k, :] * _sc_log(top[k] + 1e-16))
                     for k in range(16)]
            while len(terms) > 1:
                terms = [terms[i] + terms[i + 1]
                         for i in range(0, len(terms), 2)]
            return acc + terms[0]
        return lax.fori_loop(0, _NGROUP, group_step, acc)

    def chunk_body(ci, acc):
        pltpu.sync_copy(
            x_hbm.at[layer, :, pl.ds(t_base + ci * _CHUNK, _CHUNK)], buf)
        return process(buf, acc)

    acc = lax.fori_loop(0, _NCHUNK, chunk_body, jnp.zeros((16,), jnp.float32))
    accv[...] = acc
    pltpu.sync_copy(accv, out_hbm.at[wid])


def _make_route_call():
    mesh = plsc.VectorSubcoreMesh(core_axis_name="c", subcore_axis_name="s")
    return functools.partial(
        pl.kernel,
        out_type=jax.ShapeDtypeStruct((_NW, 16), jnp.float32),
        mesh=mesh,
        scratch_types=[
            pltpu.VMEM((_E, _CHUNK), jnp.float32),
            pltpu.VMEM((16, 16), jnp.float32),
            pltpu.VMEM((16, 16), jnp.float32),
            pltpu.VMEM((16,), jnp.float32),
        ],
        compiler_params=pltpu.CompilerParams(use_tc_tiling_on_sc=True,
                                             needs_layout_passes=False),
    )(_route_body)


def _tc_route_body(a_ref, b_ref, x_ref, out_ref):
    l = pl.program_id(0)
    x = x_ref[0]
    x3 = x.reshape(_E, _TC_BLK // 128, 128)
    regs = [x3[e] for e in range(_E)]
    s0 = _sort16_desc(regs[0:16])
    s1 = _sort16_desc(regs[16:32])
    m1 = _merge_top16(s0, s1)
    s2 = _sort16_desc(regs[32:48])
    s3 = _sort16_desc(regs[48:64])
    m2 = _merge_top16(s2, s3)
    top = _merge_top16(m1, m2)
    terms = [jnp.abs(b_ref[l, k, 0] - a_ref[l, k, 0] * jnp.log(top[k] + 1e-16))
             for k in range(16)]
    while len(terms) > 1:
        terms = [terms[i] + terms[i + 1] for i in range(0, len(terms), 2)]
    out_ref[0, 0] = terms[0]


def _tc_route_call(a, b, xt):
    return pl.pallas_call(
        _tc_route_body,
        grid=(_L, _TC_NBLK),
        in_specs=[
            pl.BlockSpec(memory_space=pltpu.SMEM),
            pl.BlockSpec(memory_space=pltpu.SMEM),
            pl.BlockSpec((1, _E, _TC_BLK), lambda l, t: (l, 0, t)),
        ],
        out_specs=pl.BlockSpec((1, 1, _TC_BLK // 128, 128),
                               lambda l, t: (l, t, 0, 0)),
        out_shape=jax.ShapeDtypeStruct((_L, _TC_NBLK, _TC_BLK // 128, 128),
                                       jnp.float32),
    )(a, b, xt)


def _sort_desc_rows(v):
    vi = v[:, :, None]
    vj = v[:, None, :]
    ii = lax.broadcasted_iota(jnp.int32, (_L, _E, _E), 1)
    jj = lax.broadcasted_iota(jnp.int32, (_L, _E, _E), 2)
    gt = (vj > vi) | ((vj == vi) & (jj < ii))
    rank = jnp.sum(gt.astype(jnp.int32), axis=2)
    rr = lax.broadcasted_iota(jnp.int32, (_L, _E, _E), 2)
    onehot = rank[:, :, None] == rr
    return jnp.sum(jnp.where(onehot, vi, 0.0), axis=1)


def _final_body(count_ref, theta_ref, part_ref, tcpart_ref, out_ref):
    count = count_ref[...]
    theta = theta_ref[...]
    x = lax.broadcasted_iota(jnp.int32, (_L, _E), 1).astype(jnp.float32)
    lp = (-0.5 * ((x - _MIU) / theta) ** 2 - jnp.log(theta)
          - np.float32(0.5 * math.log(2.0 * math.pi)))
    norm_prob = jnp.exp(lp)
    norm_prob = norm_prob / jnp.sum(norm_prob)
    s_count = _sort_desc_rows(count)
    s_np = _sort_desc_rows(norm_prob)
    temp = s_count / jnp.sum(s_count, axis=1, keepdims=True)
    lterm = s_np * jnp.abs(jnp.log(s_np / (temp + 1e-8)))
    closs = jnp.sum(lterm) / np.float32(_L)
    route = ((jnp.sum(part_ref[...]) + jnp.sum(tcpart_ref[...]))
             / np.float32(_NROWS + 1e-16))
    res = _ROUTE_COEF * route + _COUNT_COEF * closs
    out_ref[...] = lax.broadcast_in_dim(res, (1, 1), ())


_final_call = pl.pallas_call(
    _final_body,
    out_shape=jax.ShapeDtypeStruct((1, 1), jnp.float32),
)


def kernel(router_outputs, attention_mask):
    del attention_mask
    count = router_outputs[:, -3, :]
    lamb = router_outputs[:, -2, :]
    theta = router_outputs[:, -1, :]
    a, b = _pois_call(lamb, jnp.asarray(_LGAMMA_KP1)[None, :])
    xt = jnp.swapaxes(router_outputs, 1, 2)
    partials = _make_route_call()(xt, a, b)
    tc_partials = _tc_route_call(a, b, xt)
    out = _final_call(count, theta, partials, tc_partials)
    return out[0, 0]
